# Initial kernel scaffold; baseline (speedup 1.0000x reference)
import jax
import jax.numpy as jnp
from jax import lax
from jax.experimental import pallas as pl
from jax.experimental.pallas import tpu as pltpu
from jax.experimental.pallas import tpu_sc as plsc

_N = 10000
_E = 320000
_D = 128
_CH = 80
_NC = 2
_NS = 16
_NW = _NC * _NS
_EPW = _E // _NW
_CPW = _EPW // _CH
_NCHUNKS = _N // _CH


def _sc_scatter(idx2d, edge_feat):
    mesh = plsc.VectorSubcoreMesh(core_axis_name="c", subcore_axis_name="s")

    def body(idx_hbm, edge_hbm, out_hbm, idxbuf, edgebuf, agg_shared):
        cid = lax.axis_index("c")
        sid = lax.axis_index("s")
        wid = cid * _NS + sid

        zv = jnp.zeros((16,), jnp.float32)

        def zrow(i, carry):
            def zcol(j, c2):
                edgebuf[i, pl.ds(j * 16, 16)] = zv
                return c2
            return lax.fori_loop(0, _D // 16, zcol, carry)

        lax.fori_loop(0, _CH, zrow, 0)

        def zchunk(k, carry):
            c = sid + k * _NS

            @pl.when(c < _NCHUNKS)
            def _():
                pltpu.sync_copy(edgebuf, agg_shared.at[pl.ds(c * _CH, _CH)])

            return carry

        lax.fori_loop(0, (_NCHUNKS + _NS - 1) // _NS, zchunk, 0)
        plsc.subcore_barrier()

        pltpu.sync_copy(idx_hbm.at[pl.ds(wid * _CPW, _CPW)], idxbuf)

        def step(i, carry):
            pltpu.sync_copy(
                edge_hbm.at[pl.ds(wid * _EPW + i * _CH, _CH)], edgebuf)
            pltpu.sync_copy(edgebuf, agg_shared.at[idxbuf.at[i]], add=True)
            return carry

        lax.fori_loop(0, _CPW, step, 0)
        plsc.subcore_barrier()

        def dump(k, carry):
            c = sid + k * _NS

            @pl.when(c < _NCHUNKS)
            def _():
                pltpu.sync_copy(agg_shared.at[pl.ds(c * _CH, _CH)],
                                out_hbm.at[cid, pl.ds(c * _CH, _CH)])

            return carry

        lax.fori_loop(0, (_NCHUNKS + _NS - 1) // _NS, dump, 0)

    return pl.kernel(
        body,
        out_type=jax.ShapeDtypeStruct((_NC, _N, _D), jnp.float32),
        mesh=mesh,
        scratch_types=[
            pltpu.VMEM((_CPW, _CH), jnp.int32),
            pltpu.VMEM((_CH, _D), jnp.float32),
            pltpu.VMEM_SHARED((_N, _D), jnp.float32),
        ],
    )(idx2d, edge_feat)


_BR = 1000


def _node_linear(node_feat, partials, W, b):
    def body(nf, a0, a1, wt, wb, bb, o):
        agg = a0[0] + a1[0]
        acc = jnp.dot(nf[...], wt[0], preferred_element_type=jnp.float32)
        acc = acc + jnp.dot(agg, wb[0], preferred_element_type=jnp.float32)
        o[...] = acc + bb[...]

    w3 = W.reshape(2, _D, _D)
    b2 = b.reshape(1, _D)
    return pl.pallas_call(
        body,
        grid=(_N // _BR,),
        in_specs=[
            pl.BlockSpec((_BR, _D), lambda i: (i, 0)),
            pl.BlockSpec((1, _BR, _D), lambda i: (0, i, 0)),
            pl.BlockSpec((1, _BR, _D), lambda i: (1, i, 0)),
            pl.BlockSpec((1, _D, _D), lambda i: (0, 0, 0)),
            pl.BlockSpec((1, _D, _D), lambda i: (1, 0, 0)),
            pl.BlockSpec((1, _D), lambda i: (0, 0)),
        ],
        out_specs=pl.BlockSpec((_BR, _D), lambda i: (i, 0)),
        out_shape=jax.ShapeDtypeStruct((_N, _D), jnp.float32),
    )(node_feat, partials, partials, w3, w3, b2)


def kernel(node_feat, edge_feat, edge_index, W, b):
    idx2d = edge_index[0].reshape(_E // _CH, _CH)
    partials = _sc_scatter(idx2d, edge_feat)
    return _node_linear(node_feat, partials, W, b)


# speedup vs baseline: 4.4924x; 4.4924x over previous
import jax
import jax.numpy as jnp
from jax import lax
from jax.experimental import pallas as pl
from jax.experimental.pallas import tpu as pltpu
from jax.experimental.pallas import tpu_sc as plsc

_N = 10000
_E = 320000
_D = 128
_CH = 80
_NC = 2
_NS = 16
_NW = _NC * _NS
_EPW = _E // _NW
_CPW = _EPW // _CH
_NCHUNKS = _N // _CH


def _sc_scatter(idx2d, edge_feat):
    mesh = plsc.VectorSubcoreMesh(core_axis_name="c", subcore_axis_name="s")

    def body(idx_hbm, edge_hbm, out_hbm, idxbuf, edgebuf, agg_shared):
        cid = lax.axis_index("c")
        sid = lax.axis_index("s")
        wid = cid * _NS + sid

        zv = jnp.zeros((16,), jnp.float32)

        def zrow(i, carry):
            def zcol(j, c2):
                edgebuf[i, pl.ds(j * 16, 16)] = zv
                return c2
            return lax.fori_loop(0, _D // 16, zcol, carry)

        lax.fori_loop(0, _CH, zrow, 0)

        def zchunk(k, carry):
            c = sid + k * _NS

            @pl.when(c < _NCHUNKS)
            def _():
                pltpu.sync_copy(edgebuf, agg_shared.at[pl.ds(c * _CH, _CH)])

            return carry

        lax.fori_loop(0, (_NCHUNKS + _NS - 1) // _NS, zchunk, 0)
        plsc.subcore_barrier()

        pltpu.sync_copy(idx_hbm.at[wid], idxbuf)

        def step(i, carry):
            pltpu.sync_copy(
                edge_hbm.at[pl.ds(wid * _EPW + i * _CH, _CH)], edgebuf)
            pltpu.sync_copy(edgebuf, agg_shared.at[idxbuf.at[i]], add=True)
            return carry

        lax.fori_loop(0, _CPW, step, 0)
        plsc.subcore_barrier()

        def dump(k, carry):
            c = sid + k * _NS

            @pl.when(c < _NCHUNKS)
            def _():
                pltpu.sync_copy(agg_shared.at[pl.ds(c * _CH, _CH)],
                                out_hbm.at[cid, pl.ds(c * _CH, _CH)])

            return carry

        lax.fori_loop(0, (_NCHUNKS + _NS - 1) // _NS, dump, 0)

    return pl.kernel(
        body,
        out_type=jax.ShapeDtypeStruct((_NC, _N, _D), jnp.float32),
        mesh=mesh,
        scratch_types=[
            pltpu.VMEM((_CPW, _CH), jnp.int32),
            pltpu.VMEM((_CH, _D), jnp.float32),
            pltpu.VMEM_SHARED((_N, _D), jnp.float32),
        ],
    )(idx2d, edge_feat)


_BR = 1000


def _node_linear(node_feat, partials, W, b):
    def body(nf, a0, a1, wt, wb, bb, o):
        agg = a0[0] + a1[0]
        acc = jnp.dot(nf[...], wt[0], preferred_element_type=jnp.float32)
        acc = acc + jnp.dot(agg, wb[0], preferred_element_type=jnp.float32)
        o[...] = acc + bb[...]

    w3 = W.reshape(2, _D, _D)
    b2 = b.reshape(1, _D)
    return pl.pallas_call(
        body,
        grid=(_N // _BR,),
        in_specs=[
            pl.BlockSpec((_BR, _D), lambda i: (i, 0)),
            pl.BlockSpec((1, _BR, _D), lambda i: (0, i, 0)),
            pl.BlockSpec((1, _BR, _D), lambda i: (1, i, 0)),
            pl.BlockSpec((1, _D, _D), lambda i: (0, 0, 0)),
            pl.BlockSpec((1, _D, _D), lambda i: (1, 0, 0)),
            pl.BlockSpec((1, _D), lambda i: (0, 0)),
        ],
        out_specs=pl.BlockSpec((_BR, _D), lambda i: (i, 0)),
        out_shape=jax.ShapeDtypeStruct((_N, _D), jnp.float32),
    )(node_feat, partials, partials, w3, w3, b2)


def kernel(node_feat, edge_feat, edge_index, W, b):
    idx2d = edge_index[0].reshape(_NW, _CPW, _CH)
    partials = _sc_scatter(idx2d, edge_feat)
    return _node_linear(node_feat, partials, W, b)


# speedup vs baseline: 5.6231x; 1.2517x over previous
import jax
import jax.numpy as jnp
from jax import lax
from jax.experimental import pallas as pl
from jax.experimental.pallas import tpu as pltpu
from jax.experimental.pallas import tpu_sc as plsc

_N = 10000
_E = 320000
_D = 128
_CH = 80
_NC = 2
_NS = 16
_NW = _NC * _NS
_EPW = _E // _NW
_CPW = _EPW // _CH
_NCHUNKS = _N // _CH


def _sc_scatter(idx2d, edge_feat):
    mesh = plsc.VectorSubcoreMesh(core_axis_name="c", subcore_axis_name="s")

    def body(idx_hbm, edge_hbm, out_hbm, idxbuf, edgebuf, ebuf1,
             agg_shared, isem, gsem0, gsem1, ssem0, ssem1):
        cid = lax.axis_index("c")
        sid = lax.axis_index("s")
        wid = cid * _NS + sid

        pltpu.async_copy(idx_hbm.at[wid], idxbuf, isem)

        zv = jnp.zeros((16,), jnp.float32)

        def zrow(i, carry):
            def zcol(j, c2):
                edgebuf[i, pl.ds(j * 16, 16)] = zv
                return c2
            return lax.fori_loop(0, _D // 16, zcol, carry)

        lax.fori_loop(0, _CH, zrow, 0)

        def zchunk(k, carry):
            c = sid + k * _NS

            @pl.when(c < _NCHUNKS)
            def _():
                pltpu.sync_copy(edgebuf, agg_shared.at[pl.ds(c * _CH, _CH)])

            return carry

        lax.fori_loop(0, (_NCHUNKS + _NS - 1) // _NS, zchunk, 0)

        bufs = (edgebuf, ebuf1)
        gsems = (gsem0, gsem1)
        ssems = (ssem0, ssem1)

        def start_gather(i, b):
            pltpu.async_copy(
                edge_hbm.at[pl.ds(wid * _EPW + i * _CH, _CH)],
                bufs[b], gsems[b])

        def wait_gather(b):
            pltpu.make_async_copy(
                edge_hbm.at[pl.ds(0, _CH)], bufs[b], gsems[b]).wait()

        def start_scatter(i, b):
            pltpu.async_copy(
                bufs[b], agg_shared.at[idxbuf.at[i]], ssems[b], add=True)

        def wait_scatter(i, b):
            pltpu.make_async_copy(
                bufs[b], agg_shared.at[idxbuf.at[i]], ssems[b]).wait()

        start_gather(0, 0)
        start_gather(1, 1)
        plsc.subcore_barrier()
        pltpu.make_async_copy(idx_hbm.at[wid], idxbuf, isem).wait()

        def pair(g, carry):
            i0 = 2 * g
            i1 = i0 + 1
            wait_gather(0)
            start_scatter(i0, 0)

            @pl.when(i1 < _CPW)
            def _():
                wait_gather(1)
                start_scatter(i1, 1)

            wait_scatter(i0, 0)

            @pl.when(i0 + 2 < _CPW)
            def _():
                start_gather(i0 + 2, 0)

            @pl.when(i1 < _CPW)
            def _():
                wait_scatter(i1, 1)

                @pl.when(i1 + 2 < _CPW)
                def _():
                    start_gather(i1 + 2, 1)

            return carry

        lax.fori_loop(0, (_CPW + 1) // 2, pair, 0)
        plsc.subcore_barrier()

        def dump(k, carry):
            c = sid + k * _NS

            @pl.when(c < _NCHUNKS)
            def _():
                pltpu.sync_copy(agg_shared.at[pl.ds(c * _CH, _CH)],
                                out_hbm.at[cid, pl.ds(c * _CH, _CH)])

            return carry

        lax.fori_loop(0, (_NCHUNKS + _NS - 1) // _NS, dump, 0)

    return pl.kernel(
        body,
        out_type=jax.ShapeDtypeStruct((_NC, _N, _D), jnp.float32),
        mesh=mesh,
        scratch_types=[
            pltpu.VMEM((_CPW, _CH), jnp.int32),
            pltpu.VMEM((_CH, _D), jnp.float32),
            pltpu.VMEM((_CH, _D), jnp.float32),
            pltpu.VMEM_SHARED((_N, _D), jnp.float32),
            pltpu.SemaphoreType.DMA,
            pltpu.SemaphoreType.DMA,
            pltpu.SemaphoreType.DMA,
            pltpu.SemaphoreType.DMA,
            pltpu.SemaphoreType.DMA,
        ],
    )(idx2d, edge_feat)


_BR = 1000


def _node_linear(node_feat, partials, W, b):
    def body(nf, a0, a1, wt, wb, bb, o):
        agg = a0[0] + a1[0]
        acc = jnp.dot(nf[...], wt[0], preferred_element_type=jnp.float32)
        acc = acc + jnp.dot(agg, wb[0], preferred_element_type=jnp.float32)
        o[...] = acc + bb[...]

    w3 = W.reshape(2, _D, _D)
    b2 = b.reshape(1, _D)
    return pl.pallas_call(
        body,
        grid=(_N // _BR,),
        in_specs=[
            pl.BlockSpec((_BR, _D), lambda i: (i, 0)),
            pl.BlockSpec((1, _BR, _D), lambda i: (0, i, 0)),
            pl.BlockSpec((1, _BR, _D), lambda i: (1, i, 0)),
            pl.BlockSpec((1, _D, _D), lambda i: (0, 0, 0)),
            pl.BlockSpec((1, _D, _D), lambda i: (1, 0, 0)),
            pl.BlockSpec((1, _D), lambda i: (0, 0)),
        ],
        out_specs=pl.BlockSpec((_BR, _D), lambda i: (i, 0)),
        out_shape=jax.ShapeDtypeStruct((_N, _D), jnp.float32),
    )(node_feat, partials, partials, w3, w3, b2)


def kernel(node_feat, edge_feat, edge_index, W, b):
    idx2d = edge_index[0].reshape(_NW, _CPW, _CH)
    partials = _sc_scatter(idx2d, edge_feat)
    return _node_linear(node_feat, partials, W, b)


# speedup vs baseline: 6.8669x; 1.2212x over previous
import jax
import jax.numpy as jnp
from jax import lax
from jax.experimental import pallas as pl
from jax.experimental.pallas import tpu as pltpu
from jax.experimental.pallas import tpu_sc as plsc

_N = 10000
_E = 320000
_D = 128
_CH = 80
_NC = 2
_NS = 16
_NW = _NC * _NS
_EPW = _E // _NW
_CPW = _EPW // _CH
_NCHUNKS = _N // _CH


def _sc_scatter(idx2d, edge_feat):
    mesh = plsc.VectorSubcoreMesh(core_axis_name="c", subcore_axis_name="s")

    def body(idx_hbm, edge_hbm, out_hbm, idxbuf, edgebuf, ebuf1, ebuf2,
             agg_shared, isem, gsem0, gsem1, gsem2, ssem0, ssem1, ssem2):
        cid = lax.axis_index("c")
        sid = lax.axis_index("s")
        wid = cid * _NS + sid

        pltpu.async_copy(idx_hbm.at[wid], idxbuf, isem)

        zv = jnp.zeros((16,), jnp.float32)

        def zrow(i, carry):
            def zcol(j, c2):
                edgebuf[i, pl.ds(j * 16, 16)] = zv
                return c2
            return lax.fori_loop(0, _D // 16, zcol, carry)

        lax.fori_loop(0, _CH, zrow, 0)

        def zchunk(k, carry):
            c = sid + k * _NS

            @pl.when(c < _NCHUNKS)
            def _():
                pltpu.sync_copy(edgebuf, agg_shared.at[pl.ds(c * _CH, _CH)])

            return carry

        lax.fori_loop(0, (_NCHUNKS + _NS - 1) // _NS, zchunk, 0)

        bufs = (edgebuf, ebuf1, ebuf2)
        gsems = (gsem0, gsem1, gsem2)
        ssems = (ssem0, ssem1, ssem2)
        nbuf = 3

        def start_gather(i, b):
            pltpu.async_copy(
                edge_hbm.at[pl.ds(wid * _EPW + i * _CH, _CH)],
                bufs[b], gsems[b])

        def wait_gather(b):
            pltpu.make_async_copy(
                edge_hbm.at[pl.ds(0, _CH)], bufs[b], gsems[b]).wait()

        def start_scatter(i, b):
            pltpu.async_copy(
                bufs[b], agg_shared.at[idxbuf.at[i]], ssems[b], add=True)

        def wait_scatter(i, b):
            pltpu.make_async_copy(
                bufs[b], agg_shared.at[idxbuf.at[i]], ssems[b]).wait()

        for b in range(nbuf):
            start_gather(b, b)
        plsc.subcore_barrier()
        pltpu.make_async_copy(idx_hbm.at[wid], idxbuf, isem).wait()

        def group(g, carry):
            base = nbuf * g
            for b in range(nbuf):
                i = base + b

                @pl.when(i < _CPW)
                def _(i=i, b=b):
                    wait_gather(b)
                    start_scatter(i, b)

            for b in range(nbuf):
                i = base + b

                @pl.when(i < _CPW)
                def _(i=i, b=b):
                    wait_scatter(i, b)

                    @pl.when(i + nbuf < _CPW)
                    def _():
                        start_gather(i + nbuf, b)

            return carry

        lax.fori_loop(0, (_CPW + nbuf - 1) // nbuf, group, 0)
        plsc.subcore_barrier()

        def dump(k, carry):
            c = sid + k * _NS

            @pl.when(c < _NCHUNKS)
            def _():
                pltpu.sync_copy(agg_shared.at[pl.ds(c * _CH, _CH)],
                                out_hbm.at[cid, pl.ds(c * _CH, _CH)])

            return carry

        lax.fori_loop(0, (_NCHUNKS + _NS - 1) // _NS, dump, 0)

    return pl.kernel(
        body,
        out_type=jax.ShapeDtypeStruct((_NC, _N, _D), jnp.float32),
        mesh=mesh,
        scratch_types=[
            pltpu.VMEM((_CPW, _CH), jnp.int32),
            pltpu.VMEM((_CH, _D), jnp.float32),
            pltpu.VMEM((_CH, _D), jnp.float32),
            pltpu.VMEM((_CH, _D), jnp.float32),
            pltpu.VMEM_SHARED((_N, _D), jnp.float32),
        ] + [pltpu.SemaphoreType.DMA] * 7,
    )(idx2d, edge_feat)


_BR = 1000


def _node_linear(node_feat, partials, W, b):
    def body(nf, a0, a1, wt, wb, bb, o):
        agg = a0[0] + a1[0]
        acc = jnp.dot(nf[...], wt[0], preferred_element_type=jnp.float32)
        acc = acc + jnp.dot(agg, wb[0], preferred_element_type=jnp.float32)
        o[...] = acc + bb[...]

    w3 = W.reshape(2, _D, _D)
    b2 = b.reshape(1, _D)
    return pl.pallas_call(
        body,
        grid=(_N // _BR,),
        in_specs=[
            pl.BlockSpec((_BR, _D), lambda i: (i, 0)),
            pl.BlockSpec((1, _BR, _D), lambda i: (0, i, 0)),
            pl.BlockSpec((1, _BR, _D), lambda i: (1, i, 0)),
            pl.BlockSpec((1, _D, _D), lambda i: (0, 0, 0)),
            pl.BlockSpec((1, _D, _D), lambda i: (1, 0, 0)),
            pl.BlockSpec((1, _D), lambda i: (0, 0)),
        ],
        out_specs=pl.BlockSpec((_BR, _D), lambda i: (i, 0)),
        out_shape=jax.ShapeDtypeStruct((_N, _D), jnp.float32),
    )(node_feat, partials, partials, w3, w3, b2)


def kernel(node_feat, edge_feat, edge_index, W, b):
    idx2d = edge_index[0].reshape(_NW, _CPW, _CH)
    partials = _sc_scatter(idx2d, edge_feat)
    return _node_linear(node_feat, partials, W, b)


# speedup vs baseline: 7.3908x; 1.0763x over previous
import jax
import jax.numpy as jnp
from jax import lax
from jax.experimental import pallas as pl
from jax.experimental.pallas import tpu as pltpu
from jax.experimental.pallas import tpu_sc as plsc

_N = 10000
_E = 320000
_D = 128
_CH = 80
_NC = 2
_NS = 16
_NW = _NC * _NS
_EPW = _E // _NW
_CPW = _EPW // _CH
_NCHUNKS = _N // _CH


def _sc_scatter(idx2d, edge_feat):
    mesh = plsc.VectorSubcoreMesh(core_axis_name="c", subcore_axis_name="s")

    def body(idx_hbm, edge_hbm, out_hbm, idxbuf, edgebuf, ebuf1, ebuf2,
             ebuf3, agg_shared, isem, gsem0, gsem1, gsem2, gsem3, ssem0,
             ssem1, ssem2, ssem3):
        cid = lax.axis_index("c")
        sid = lax.axis_index("s")
        wid = cid * _NS + sid

        pltpu.async_copy(idx_hbm.at[pl.ds(wid * _EPW, _EPW)], idxbuf, isem)

        zv = jnp.zeros((16,), jnp.float32)

        def zrow(i, carry):
            def zcol(j, c2):
                edgebuf[i, pl.ds(j * 16, 16)] = zv
                return c2
            return lax.fori_loop(0, _D // 16, zcol, carry)

        lax.fori_loop(0, _CH, zrow, 0)

        def zchunk(k, carry):
            c = sid + k * _NS

            @pl.when(c < _NCHUNKS)
            def _():
                pltpu.sync_copy(edgebuf, agg_shared.at[pl.ds(c * _CH, _CH)])

            return carry

        lax.fori_loop(0, (_NCHUNKS + _NS - 1) // _NS, zchunk, 0)

        bufs = (edgebuf, ebuf1, ebuf2, ebuf3)
        gsems = (gsem0, gsem1, gsem2, gsem3)
        ssems = (ssem0, ssem1, ssem2, ssem3)
        nbuf = 4

        def start_gather(i, b):
            pltpu.async_copy(
                edge_hbm.at[pl.ds(wid * _EPW + i * _CH, _CH)],
                bufs[b], gsems[b])

        def wait_gather(b):
            pltpu.make_async_copy(
                edge_hbm.at[pl.ds(0, _CH)], bufs[b], gsems[b]).wait()

        def start_scatter(i, b):
            pltpu.async_copy(
                bufs[b], agg_shared.at[idxbuf.at[pl.ds(i * _CH, _CH)]],
                ssems[b], add=True)

        def wait_scatter(i, b):
            pltpu.make_async_copy(
                bufs[b], agg_shared.at[idxbuf.at[pl.ds(i * _CH, _CH)]],
                ssems[b]).wait()

        for b in range(nbuf):
            start_gather(b, b)
        plsc.subcore_barrier()
        pltpu.make_async_copy(
            idx_hbm.at[pl.ds(wid * _EPW, _EPW)], idxbuf, isem).wait()

        def group(g, carry):
            base = nbuf * g
            for b in range(nbuf):
                i = base + b

                @pl.when(i < _CPW)
                def _(i=i, b=b):
                    wait_gather(b)
                    start_scatter(i, b)

            for b in range(nbuf):
                i = base + b

                @pl.when(i < _CPW)
                def _(i=i, b=b):
                    wait_scatter(i, b)

                    @pl.when(i + nbuf < _CPW)
                    def _():
                        start_gather(i + nbuf, b)

            return carry

        lax.fori_loop(0, (_CPW + nbuf - 1) // nbuf, group, 0)
        plsc.subcore_barrier()

        def dump(k, carry):
            c = sid + k * _NS

            @pl.when(c < _NCHUNKS)
            def _():
                pltpu.sync_copy(agg_shared.at[pl.ds(c * _CH, _CH)],
                                out_hbm.at[cid, pl.ds(c * _CH, _CH)])

            return carry

        lax.fori_loop(0, (_NCHUNKS + _NS - 1) // _NS, dump, 0)

    return pl.kernel(
        body,
        out_type=jax.ShapeDtypeStruct((_NC, _N, _D), jnp.float32),
        mesh=mesh,
        scratch_types=[
            pltpu.VMEM((_EPW,), jnp.int32),
            pltpu.VMEM((_CH, _D), jnp.float32),
            pltpu.VMEM((_CH, _D), jnp.float32),
            pltpu.VMEM((_CH, _D), jnp.float32),
            pltpu.VMEM((_CH, _D), jnp.float32),
            pltpu.VMEM_SHARED((_N, _D), jnp.float32),
        ] + [pltpu.SemaphoreType.DMA] * 9,
    )(idx2d, edge_feat)


_BR = 1000


def _node_linear(node_feat, partials, W, b):
    def body(nf, a0, a1, wt, wb, bb, o):
        agg = a0[0] + a1[0]
        acc = jnp.dot(nf[...], wt[0], preferred_element_type=jnp.float32)
        acc = acc + jnp.dot(agg, wb[0], preferred_element_type=jnp.float32)
        o[...] = acc + bb[...]

    w3 = W.reshape(2, _D, _D)
    b2 = b.reshape(1, _D)
    return pl.pallas_call(
        body,
        grid=(_N // _BR,),
        in_specs=[
            pl.BlockSpec((_BR, _D), lambda i: (i, 0)),
            pl.BlockSpec((1, _BR, _D), lambda i: (0, i, 0)),
            pl.BlockSpec((1, _BR, _D), lambda i: (1, i, 0)),
            pl.BlockSpec((1, _D, _D), lambda i: (0, 0, 0)),
            pl.BlockSpec((1, _D, _D), lambda i: (1, 0, 0)),
            pl.BlockSpec((1, _D), lambda i: (0, 0)),
        ],
        out_specs=pl.BlockSpec((_BR, _D), lambda i: (i, 0)),
        out_shape=jax.ShapeDtypeStruct((_N, _D), jnp.float32),
    )(node_feat, partials, partials, w3, w3, b2)


def kernel(node_feat, edge_feat, edge_index, W, b):
    idx1d = edge_index[0]
    partials = _sc_scatter(idx1d, edge_feat)
    return _node_linear(node_feat, partials, W, b)


# speedup vs baseline: 7.4727x; 1.0111x over previous
import jax
import jax.numpy as jnp
from jax import lax
from jax.experimental import pallas as pl
from jax.experimental.pallas import tpu as pltpu
from jax.experimental.pallas import tpu_sc as plsc

_N = 10000
_E = 320000
_D = 128
_CH = 80
_NC = 2
_NS = 16
_NW = _NC * _NS
_EPW = _E // _NW
_CPW = _EPW // _CH
_NCHUNKS = _N // _CH


def _sc_scatter(idx2d, edge_feat):
    mesh = plsc.VectorSubcoreMesh(core_axis_name="c", subcore_axis_name="s")

    def body(idx_hbm, edge_hbm, out_hbm, idxbuf, edgebuf, ebuf1, ebuf2,
             ebuf3, agg_shared, isem, gsem0, gsem1, gsem2, gsem3, ssem0,
             ssem1, ssem2, ssem3):
        cid = lax.axis_index("c")
        sid = lax.axis_index("s")
        wid = cid * _NS + sid

        pltpu.async_copy(idx_hbm.at[pl.ds(wid * _EPW, _EPW)], idxbuf, isem)

        for _pb in (1, 2, 3):
            pltpu.async_copy(
                edge_hbm.at[pl.ds(wid * _EPW + _pb * _CH, _CH)],
                (edgebuf, ebuf1, ebuf2, ebuf3)[_pb],
                (gsem0, gsem1, gsem2, gsem3)[_pb])

        zv = jnp.zeros((16,), jnp.float32)

        def zrow(i, carry):
            def zcol(j, c2):
                edgebuf[i, pl.ds(j * 16, 16)] = zv
                return c2
            return lax.fori_loop(0, _D // 16, zcol, carry)

        lax.fori_loop(0, _CH, zrow, 0)

        def zchunk(k, carry):
            c = sid + k * _NS

            @pl.when(c < _NCHUNKS)
            def _():
                pltpu.sync_copy(edgebuf, agg_shared.at[pl.ds(c * _CH, _CH)])

            return carry

        lax.fori_loop(0, (_NCHUNKS + _NS - 1) // _NS, zchunk, 0)

        bufs = (edgebuf, ebuf1, ebuf2, ebuf3)
        gsems = (gsem0, gsem1, gsem2, gsem3)
        ssems = (ssem0, ssem1, ssem2, ssem3)
        nbuf = 4

        def start_gather(i, b):
            pltpu.async_copy(
                edge_hbm.at[pl.ds(wid * _EPW + i * _CH, _CH)],
                bufs[b], gsems[b])

        def wait_gather(b):
            pltpu.make_async_copy(
                edge_hbm.at[pl.ds(0, _CH)], bufs[b], gsems[b]).wait()

        def start_scatter(i, b):
            pltpu.async_copy(
                bufs[b], agg_shared.at[idxbuf.at[pl.ds(i * _CH, _CH)]],
                ssems[b], add=True)

        def wait_scatter(i, b):
            pltpu.make_async_copy(
                bufs[b], agg_shared.at[idxbuf.at[pl.ds(i * _CH, _CH)]],
                ssems[b]).wait()

        start_gather(0, 0)
        plsc.subcore_barrier()
        pltpu.make_async_copy(
            idx_hbm.at[pl.ds(wid * _EPW, _EPW)], idxbuf, isem).wait()

        def group(g, carry):
            base = nbuf * g
            for b in range(nbuf):
                i = base + b

                @pl.when(i < _CPW)
                def _(i=i, b=b):
                    wait_gather(b)
                    start_scatter(i, b)

            for b in range(nbuf):
                i = base + b

                @pl.when(i < _CPW)
                def _(i=i, b=b):
                    wait_scatter(i, b)

                    @pl.when(i + nbuf < _CPW)
                    def _():
                        start_gather(i + nbuf, b)

            return carry

        lax.fori_loop(0, (_CPW + nbuf - 1) // nbuf, group, 0)
        plsc.subcore_barrier()

        def dump(k, carry):
            c = sid + k * _NS

            @pl.when(c < _NCHUNKS)
            def _():
                pltpu.sync_copy(agg_shared.at[pl.ds(c * _CH, _CH)],
                                out_hbm.at[cid, pl.ds(c * _CH, _CH)])

            return carry

        lax.fori_loop(0, (_NCHUNKS + _NS - 1) // _NS, dump, 0)

    return pl.kernel(
        body,
        out_type=jax.ShapeDtypeStruct((_NC, _N, _D), jnp.float32),
        mesh=mesh,
        scratch_types=[
            pltpu.VMEM((_EPW,), jnp.int32),
            pltpu.VMEM((_CH, _D), jnp.float32),
            pltpu.VMEM((_CH, _D), jnp.float32),
            pltpu.VMEM((_CH, _D), jnp.float32),
            pltpu.VMEM((_CH, _D), jnp.float32),
            pltpu.VMEM_SHARED((_N, _D), jnp.float32),
        ] + [pltpu.SemaphoreType.DMA] * 9,
    )(idx2d, edge_feat)


_BR = 1000


def _node_half(node_feat, W, b):
    def body(nf, wt, bb, o):
        o[...] = jnp.dot(nf[...], wt[0],
                         preferred_element_type=jnp.float32) + bb[...]

    w3 = W.reshape(2, _D, _D)
    b2 = b.reshape(1, _D)
    return pl.pallas_call(
        body,
        grid=(_N // _BR,),
        in_specs=[
            pl.BlockSpec((_BR, _D), lambda i: (i, 0)),
            pl.BlockSpec((1, _D, _D), lambda i: (0, 0, 0)),
            pl.BlockSpec((1, _D), lambda i: (0, 0)),
        ],
        out_specs=pl.BlockSpec((_BR, _D), lambda i: (i, 0)),
        out_shape=jax.ShapeDtypeStruct((_N, _D), jnp.float32),
    )(node_feat, w3, b2)


def _agg_half(tmp, partials, W):
    def body(tp, a0, a1, wb, o):
        agg = a0[0] + a1[0]
        o[...] = tp[...] + jnp.dot(agg, wb[0],
                                   preferred_element_type=jnp.float32)

    w3 = W.reshape(2, _D, _D)
    return pl.pallas_call(
        body,
        grid=(_N // _BR,),
        in_specs=[
            pl.BlockSpec((_BR, _D), lambda i: (i, 0)),
            pl.BlockSpec((1, _BR, _D), lambda i: (0, i, 0)),
            pl.BlockSpec((1, _BR, _D), lambda i: (1, i, 0)),
            pl.BlockSpec((1, _D, _D), lambda i: (1, 0, 0)),
        ],
        out_specs=pl.BlockSpec((_BR, _D), lambda i: (i, 0)),
        out_shape=jax.ShapeDtypeStruct((_N, _D), jnp.float32),
    )(tmp, partials, partials, w3)


def kernel(node_feat, edge_feat, edge_index, W, b):
    idx1d = edge_index[0]
    partials = _sc_scatter(idx1d, edge_feat)
    tmp = _node_half(node_feat, W, b)
    return _agg_half(tmp, partials, W)


# speedup vs baseline: 7.9328x; 1.0616x over previous
import jax
import jax.numpy as jnp
from jax import lax
from jax.experimental import pallas as pl
from jax.experimental.pallas import tpu as pltpu
from jax.experimental.pallas import tpu_sc as plsc

_N = 10000
_E = 320000
_D = 128
_CH = 128
_NC = 2
_NS = 16
_NW = _NC * _NS
_NB = _E // _CH
_BPW = _NB // _NW
_EXTRA = _NB % _NW
_BMAX = _BPW + 1
_ZCH = 80
_NZ = _N // _ZCH
_NBUF = 3


def _sc_scatter(edge_index, edge_feat):
    mesh = plsc.VectorSubcoreMesh(core_axis_name="c", subcore_axis_name="s")

    def body(eidx_hbm, edge_hbm, out_hbm,
             ibuf0, ibuf1, ibuf2, ebuf0, ebuf1, ebuf2, agg_shared,
             gsem0, gsem1, gsem2, isem0, isem1, isem2,
             ssem0, ssem1, ssem2):
        cid = lax.axis_index("c")
        sid = lax.axis_index("s")
        wid = cid * _NS + sid
        nblk = _BPW + jnp.where(wid < _EXTRA, 1, 0)
        blk0 = _BPW * wid + jnp.minimum(wid, _EXTRA)

        ibufs = (ibuf0, ibuf1, ibuf2)
        ebufs = (ebuf0, ebuf1, ebuf2)
        gsems = (gsem0, gsem1, gsem2)
        isems = (isem0, isem1, isem2)
        ssems = (ssem0, ssem1, ssem2)

        def start_gather(i, b):
            off = (blk0 + i) * _CH
            pltpu.async_copy(edge_hbm.at[pl.ds(off, _CH)], ebufs[b],
                             gsems[b])
            pltpu.async_copy(eidx_hbm.at[0, pl.ds(off, _CH)], ibufs[b],
                             isems[b])

        def wait_gather(b):
            pltpu.make_async_copy(
                edge_hbm.at[pl.ds(0, _CH)], ebufs[b], gsems[b]).wait()
            pltpu.make_async_copy(
                eidx_hbm.at[0, pl.ds(0, _CH)], ibufs[b], isems[b]).wait()

        def start_scatter(b):
            pltpu.async_copy(ebufs[b], agg_shared.at[ibufs[b]], ssems[b],
                             add=True)

        def wait_scatter(b):
            pltpu.make_async_copy(
                ebufs[b], agg_shared.at[ibufs[b]], ssems[b]).wait()

        for pb in (1, 2):
            start_gather(pb, pb)

        zv = jnp.zeros((16,), jnp.float32)

        def zrow(i, carry):
            def zcol(j, c2):
                ebuf0[i, pl.ds(j * 16, 16)] = zv
                return c2
            return lax.fori_loop(0, _D // 16, zcol, carry)

        lax.fori_loop(0, _ZCH, zrow, 0)

        def zchunk(k, carry):
            c = sid + k * _NS

            @pl.when(c < _NZ)
            def _():
                pltpu.sync_copy(ebuf0.at[pl.ds(0, _ZCH)],
                                agg_shared.at[pl.ds(c * _ZCH, _ZCH)])

            return carry

        lax.fori_loop(0, (_NZ + _NS - 1) // _NS, zchunk, 0)
        start_gather(0, 0)
        plsc.subcore_barrier()

        def group(g, carry):
            base = _NBUF * g
            for b in range(_NBUF):
                i = base + b

                @pl.when(i < nblk)
                def _(i=i, b=b):
                    wait_gather(b)
                    start_scatter(b)

            for b in range(_NBUF):
                i = base + b

                @pl.when(i < nblk)
                def _(i=i, b=b):
                    wait_scatter(b)

                    @pl.when(i + _NBUF < nblk)
                    def _():
                        start_gather(i + _NBUF, b)

            return carry

        lax.fori_loop(0, (_BMAX + _NBUF - 1) // _NBUF, group, 0)
        plsc.subcore_barrier()

        def dump(k, carry):
            c = sid + k * _NS

            @pl.when(c < _NZ)
            def _():
                pltpu.sync_copy(agg_shared.at[pl.ds(c * _ZCH, _ZCH)],
                                out_hbm.at[cid, pl.ds(c * _ZCH, _ZCH)])

            return carry

        lax.fori_loop(0, (_NZ + _NS - 1) // _NS, dump, 0)

    return pl.kernel(
        body,
        out_type=jax.ShapeDtypeStruct((_NC, _N, _D), jnp.float32),
        mesh=mesh,
        scratch_types=[
            pltpu.VMEM((_CH,), jnp.int32),
            pltpu.VMEM((_CH,), jnp.int32),
            pltpu.VMEM((_CH,), jnp.int32),
            pltpu.VMEM((_CH, _D), jnp.float32),
            pltpu.VMEM((_CH, _D), jnp.float32),
            pltpu.VMEM((_CH, _D), jnp.float32),
            pltpu.VMEM_SHARED((_N, _D), jnp.float32),
        ] + [pltpu.SemaphoreType.DMA] * 9,
    )(edge_index, edge_feat)


_BR = 2000


def _node_half(node_feat, W, b):
    def body(nf, wt, bb, o):
        o[...] = jnp.dot(nf[...], wt[0],
                         preferred_element_type=jnp.float32) + bb[...]

    w3 = W.reshape(2, _D, _D)
    b2 = b.reshape(1, _D)
    return pl.pallas_call(
        body,
        grid=(_N // _BR,),
        in_specs=[
            pl.BlockSpec((_BR, _D), lambda i: (i, 0)),
            pl.BlockSpec((1, _D, _D), lambda i: (0, 0, 0)),
            pl.BlockSpec((1, _D), lambda i: (0, 0)),
        ],
        out_specs=pl.BlockSpec((_BR, _D), lambda i: (i, 0)),
        out_shape=jax.ShapeDtypeStruct((_N, _D), jnp.float32),
    )(node_feat, w3, b2)


def _agg_half(tmp, partials, W):
    def body(tp, a0, a1, wb, o):
        agg = a0[0] + a1[0]
        o[...] = tp[...] + jnp.dot(agg, wb[0],
                                   preferred_element_type=jnp.float32)

    w3 = W.reshape(2, _D, _D)
    return pl.pallas_call(
        body,
        grid=(_N // _BR,),
        in_specs=[
            pl.BlockSpec((_BR, _D), lambda i: (i, 0)),
            pl.BlockSpec((1, _BR, _D), lambda i: (0, i, 0)),
            pl.BlockSpec((1, _BR, _D), lambda i: (1, i, 0)),
            pl.BlockSpec((1, _D, _D), lambda i: (1, 0, 0)),
        ],
        out_specs=pl.BlockSpec((_BR, _D), lambda i: (i, 0)),
        out_shape=jax.ShapeDtypeStruct((_N, _D), jnp.float32),
    )(tmp, partials, partials, w3)


def kernel(node_feat, edge_feat, edge_index, W, b):
    partials = _sc_scatter(edge_index, edge_feat)
    tmp = _node_half(node_feat, W, b)
    return _agg_half(tmp, partials, W)
